# baseline (speedup 1.0000x reference)
import jax
import jax.numpy as jnp
from jax import lax
from jax.experimental import pallas as pl
from jax.experimental.pallas import tpu as pltpu
from jax.experimental.pallas import tpu_sc as plsc

N = 100000
D = 512
H = 256
G = 1024
NC = 2
NS = 16
NW = NC * NS
GPT = G // NW

NCHUNKS = 2
CN = N // NCHUNKS

BN = 10000


def _score_body(x_ref, w1_ref, b1_ref, w2_ref, b2_ref, out_ref):
    h = jnp.tanh(
        jnp.dot(x_ref[...], w1_ref[...], preferred_element_type=jnp.float32)
        + b1_ref[...]
    )
    s = jnp.dot(h, w2_ref[...], preferred_element_type=jnp.float32)
    out_ref[...] = jnp.broadcast_to(jnp.exp(s + b2_ref[0]), (s.shape[0], 16))


def _make_score(lo):
    blk0 = lo // BN
    return pl.pallas_call(
        _score_body,
        grid=(CN // BN,),
        in_specs=[
            pl.BlockSpec((BN, D), lambda i: (i + blk0, 0)),
            pl.BlockSpec((D, H), lambda i: (0, 0)),
            pl.BlockSpec((1, H), lambda i: (0, 0)),
            pl.BlockSpec((H, 1), lambda i: (0, 0)),
            pl.BlockSpec(memory_space=pltpu.SMEM),
        ],
        out_specs=pl.BlockSpec((BN, 16), lambda i: (i, 0)),
        out_shape=jax.ShapeDtypeStruct((CN, 16), jnp.float32),
    )


_score_calls = [_make_score(c * CN) for c in range(NCHUNKS)]

K = 72
KB = K + 8
NB = 34


def _make_pool(lo):
    hi = lo + CN

    def _pool_body(x_hbm, e_hbm, rs_hbm, out_num, out_den, x_v0, x_v1, e_v0,
                   e_v1, rs_v, acc_v, den_v, rs_s, sem0, sem1):
        cid = lax.axis_index("c")
        sid = lax.axis_index("s")
        wid = sid * NC + cid

        pltpu.sync_copy(rs_hbm.at[pl.ds(wid * GPT, 48)], rs_v)
        for v in range(3):
            vec = jnp.clip(rs_v[pl.ds(v * 16, 16)], lo, hi)
            for l in range(16):
                i = v * 16 + l
                if i < NB:
                    rs_s[i] = vec[l]

        a0 = rs_s[0]
        b0 = rs_s[GPT]
        nch = (b0 - a0 + (K - 1)) // K

        zv = jnp.zeros((16,), jnp.float32)

        def _z(g, c):
            for j in range(D // 16):
                acc_v[g, pl.ds(j * 16, 16)] = zv
            den_v[g] = zv
            return c

        lax.fori_loop(0, GPT, _z, 0)

        def _base(c):
            base = a0 + c * K
            basec = jnp.minimum((base // 8) * 8, hi - KB)
            return base, basec

        def _prefetch(c, xbuf, ebuf, sem):
            _, basec = _base(c)
            pltpu.async_copy(x_hbm.at[pl.ds(basec, KB)], xbuf, sem)
            pltpu.async_copy(e_hbm.at[pl.ds(basec - lo, KB)], ebuf, sem)

        def _wait(xbuf, ebuf, sem):
            pltpu.make_async_copy(x_hbm.at[pl.ds(0, KB)], xbuf, sem).wait()
            pltpu.make_async_copy(e_hbm.at[pl.ds(0, KB)], ebuf, sem).wait()

        def _consume(c, xbuf, ebuf):
            base, basec = _base(c)
            off = base - basec

            def _row(r, st2):
                gr = basec + r
                ok = jnp.logical_and(jnp.logical_and(r >= off, r < off + K),
                                     gr < b0)
                lo_i = jnp.int32(0)
                for w in (16, 8, 4, 2, 1):
                    m = lo_i + w
                    lo_i = jnp.where(rs_s[m] <= gr, m, lo_i)
                er = jnp.where(ok, ebuf[r], 0.0)
                prods = [xbuf[r, pl.ds(j * 16, 16)] * er
                         for j in range(D // 16)]
                for j in range(D // 16):
                    plsc.addupdate(acc_v.at[lo_i, pl.ds(j * 16, 16)],
                                   prods[j])
                plsc.addupdate(den_v.at[lo_i], er)
                return st2

            lax.fori_loop(0, KB, _row, 0)

        _prefetch(0, x_v0, e_v0, sem0)
        npairs = (nch + 1) // 2

        def _pair(p, st):
            c0 = 2 * p
            _wait(x_v0, e_v0, sem0)
            _prefetch(c0 + 1, x_v1, e_v1, sem1)
            _consume(c0, x_v0, e_v0)
            _wait(x_v1, e_v1, sem1)
            _prefetch(c0 + 2, x_v0, e_v0, sem0)
            _consume(c0 + 1, x_v1, e_v1)
            return st

        _ = lax.fori_loop(0, npairs, _pair, jnp.int32(0))
        _wait(x_v0, e_v0, sem0)

        rbase = wid * GPT
        pltpu.sync_copy(acc_v, out_num.at[pl.ds(rbase, GPT)])
        pltpu.sync_copy(den_v, out_den.at[pl.ds(rbase, GPT)])

    return pl.kernel(
        _pool_body,
        out_type=(jax.ShapeDtypeStruct((G, D), jnp.float32),
                  jax.ShapeDtypeStruct((G, 16), jnp.float32)),
        mesh=plsc.VectorSubcoreMesh(core_axis_name="c", subcore_axis_name="s",
                                    num_cores=NC, num_subcores=NS),
        scratch_types=[
            pltpu.VMEM((KB, D), jnp.float32),
            pltpu.VMEM((KB, D), jnp.float32),
            pltpu.VMEM((KB, 16), jnp.float32),
            pltpu.VMEM((KB, 16), jnp.float32),
            pltpu.VMEM((48,), jnp.int32),
            pltpu.VMEM((GPT, D), jnp.float32),
            pltpu.VMEM((GPT, 16), jnp.float32),
            pltpu.SMEM((48,), jnp.int32),
            pltpu.SemaphoreType.DMA,
            pltpu.SemaphoreType.DMA,
        ],
    )


_pool_calls = [_make_pool(c * CN) for c in range(NCHUNKS)]


def _merge_body(*refs):
    num_refs = refs[:NCHUNKS]
    den_refs = refs[NCHUNKS:2 * NCHUNKS]
    out_ref = refs[2 * NCHUNKS]
    num = num_refs[0][...]
    den = den_refs[0][:, 0:1]
    for c in range(1, NCHUNKS):
        num = num + num_refs[c][...]
        den = den + den_refs[c][:, 0:1]
    den = den + jnp.where(den <= 0.0, 1.0, 0.0)
    out_ref[...] = num / den


_merge_call = pl.pallas_call(
    _merge_body,
    out_shape=jax.ShapeDtypeStruct((G, D), jnp.float32),
)


def kernel(x, batch_idx, num_graphs, W1, b1, W2, b2):
    bidx = (batch_idx
            + (jnp.asarray(num_graphs, batch_idx.dtype) - G)).astype(jnp.int32)
    rs = jnp.searchsorted(bidx, jnp.arange(G + 1, dtype=jnp.int32),
                          side="left").astype(jnp.int32)
    rs_pad = jnp.concatenate([rs, jnp.full((15,), N, jnp.int32)])
    b1r = jnp.reshape(b1, (1, H))
    b2r = jnp.reshape(b2, (1,))
    nums, dens = [], []
    for c in range(NCHUNKS):
        e_c = _score_calls[c](x, W1, b1r, W2, b2r)
        n_c, d_c = _pool_calls[c](x, e_c, rs_pad)
        nums.append(n_c)
        dens.append(d_c)
    return _merge_call(*nums, *dens)


# speedup vs baseline: 1.3712x; 1.3712x over previous
import jax
import jax.numpy as jnp
from jax import lax
from jax.experimental import pallas as pl
from jax.experimental.pallas import tpu as pltpu
from jax.experimental.pallas import tpu_sc as plsc

N = 100000
D = 512
H = 256
G = 1024
NC = 2
NS = 16
NW = NC * NS
GPT = G // NW

BN = 10000


def _score_body(x_ref, w1_ref, b1_ref, w2_ref, b2_ref, out_ref):
    h = jnp.tanh(
        jnp.dot(x_ref[...], w1_ref[...], preferred_element_type=jnp.float32)
        + b1_ref[...]
    )
    s = jnp.dot(h, w2_ref[...], preferred_element_type=jnp.float32)
    out_ref[...] = jnp.broadcast_to(jnp.exp(s + b2_ref[0]), (s.shape[0], 16))


_score_call = pl.pallas_call(
    _score_body,
    grid=(N // BN,),
    in_specs=[
        pl.BlockSpec((BN, D), lambda i: (i, 0)),
        pl.BlockSpec((D, H), lambda i: (0, 0)),
        pl.BlockSpec((1, H), lambda i: (0, 0)),
        pl.BlockSpec((H, 1), lambda i: (0, 0)),
        pl.BlockSpec(memory_space=pltpu.SMEM),
    ],
    out_specs=pl.BlockSpec((BN, 16), lambda i: (i, 0)),
    out_shape=jax.ShapeDtypeStruct((N, 16), jnp.float32),
)

K = 72
KB = K + 8
NB = 34


def _pool_body(x_hbm, e_hbm, rs_hbm, out_hbm, x_v0, x_v1, e_v0, e_v1, rs_v,
               acc_v, den_v, rs_s, sem0, sem1):
    cid = lax.axis_index("c")
    sid = lax.axis_index("s")
    wid = sid * NC + cid

    pltpu.sync_copy(rs_hbm.at[pl.ds(wid * GPT, 48)], rs_v)
    for v in range(3):
        vec = rs_v[pl.ds(v * 16, 16)]
        for l in range(16):
            i = v * 16 + l
            if i < NB:
                rs_s[i] = vec[l]

    a0 = rs_s[0]
    b0 = rs_s[GPT]
    nch = (b0 - a0 + (K - 1)) // K

    zv = jnp.zeros((16,), jnp.float32)

    def _z(g, c):
        for j in range(D // 16):
            acc_v[g, pl.ds(j * 16, 16)] = zv
        den_v[g] = zv
        return c

    lax.fori_loop(0, GPT, _z, 0)

    def _base(c):
        base = a0 + c * K
        basec = jnp.minimum((base // 8) * 8, N - KB)
        return base, basec

    def _prefetch(c, xbuf, ebuf, sem):
        _, basec = _base(c)
        pltpu.async_copy(x_hbm.at[pl.ds(basec, KB)], xbuf, sem)
        pltpu.async_copy(e_hbm.at[pl.ds(basec, KB)], ebuf, sem)

    def _wait(xbuf, ebuf, sem):
        pltpu.make_async_copy(x_hbm.at[pl.ds(0, KB)], xbuf, sem).wait()
        pltpu.make_async_copy(e_hbm.at[pl.ds(0, KB)], ebuf, sem).wait()

    def _consume(c, xbuf, ebuf):
        base, basec = _base(c)
        off = base - basec

        def _row(r, st2):
            gr = basec + r
            ok = jnp.logical_and(jnp.logical_and(r >= off, r < off + K),
                                 gr < b0)
            lo = jnp.int32(0)
            for w in (16, 8, 4, 2, 1):
                m = lo + w
                lo = jnp.where(rs_s[m] <= gr, m, lo)
            er = jnp.where(ok, ebuf[r], 0.0)
            half = (D // 16) // 2
            lds0 = [xbuf[r, pl.ds(j * 16, 16)] for j in range(half)]
            prods0 = [v * er for v in lds0]
            lds1 = [xbuf[r, pl.ds((half + j) * 16, 16)] for j in range(half)]
            for j in range(half):
                plsc.addupdate(acc_v.at[lo, pl.ds(j * 16, 16)], prods0[j])
            prods1 = [v * er for v in lds1]
            for j in range(half):
                plsc.addupdate(acc_v.at[lo, pl.ds((half + j) * 16, 16)],
                               prods1[j])
            plsc.addupdate(den_v.at[lo], er)
            return st2

        lax.fori_loop(0, KB, _row, 0)

    _prefetch(0, x_v0, e_v0, sem0)
    npairs = (nch + 1) // 2

    def _pair(p, st):
        c0 = 2 * p
        _wait(x_v0, e_v0, sem0)
        _prefetch(c0 + 1, x_v1, e_v1, sem1)
        _consume(c0, x_v0, e_v0)
        _wait(x_v1, e_v1, sem1)
        _prefetch(c0 + 2, x_v0, e_v0, sem0)
        _consume(c0 + 1, x_v1, e_v1)
        return st

    _ = lax.fori_loop(0, npairs, _pair, jnp.int32(0))
    _wait(x_v0, e_v0, sem0)

    one = jnp.ones((16,), jnp.float32)

    def _div(g, c):
        den = den_v[g]
        safe = den + jnp.where(den <= 0.0, one, zv)
        inv = one / safe
        for j in range(D // 16):
            acc_v[g, pl.ds(j * 16, 16)] = acc_v[g, pl.ds(j * 16, 16)] * inv
        return c

    lax.fori_loop(0, GPT, _div, 0)
    pltpu.sync_copy(acc_v, out_hbm.at[pl.ds(wid * GPT, GPT)])


_pool_call = pl.kernel(
    _pool_body,
    out_type=jax.ShapeDtypeStruct((G, D), jnp.float32),
    mesh=plsc.VectorSubcoreMesh(core_axis_name="c", subcore_axis_name="s",
                                num_cores=NC, num_subcores=NS),
    scratch_types=[
        pltpu.VMEM((KB, D), jnp.float32),
        pltpu.VMEM((KB, D), jnp.float32),
        pltpu.VMEM((KB, 16), jnp.float32),
        pltpu.VMEM((KB, 16), jnp.float32),
        pltpu.VMEM((48,), jnp.int32),
        pltpu.VMEM((GPT, D), jnp.float32),
        pltpu.VMEM((GPT, 16), jnp.float32),
        pltpu.SMEM((48,), jnp.int32),
        pltpu.SemaphoreType.DMA,
        pltpu.SemaphoreType.DMA,
    ],
)


def kernel(x, batch_idx, num_graphs, W1, b1, W2, b2):
    bidx = (batch_idx
            + (jnp.asarray(num_graphs, batch_idx.dtype) - G)).astype(jnp.int32)
    e = _score_call(x, W1, jnp.reshape(b1, (1, H)), W2, jnp.reshape(b2, (1,)))
    rs = jnp.searchsorted(bidx, jnp.arange(G + 1, dtype=jnp.int32),
                          side="left").astype(jnp.int32)
    rs_pad = jnp.concatenate([rs, jnp.full((15,), N, jnp.int32)])
    return _pool_call(x, e, rs_pad)
